# baseline (speedup 1.0000x reference)
import functools
import jax
import jax.numpy as jnp
from jax import lax
from jax.experimental import pallas as pl
from jax.experimental.pallas import tpu as pltpu
from jax.experimental.pallas import tpu_sc as plsc

HIDDEN = 64
NC = 2
NS = 16
NW = NC * NS
TILE_B = 128


def _transpose_pad(weight):
    vocab, hidden = weight.shape
    wt = weight.T
    vb = 2048
    grid = (vocab + vb - 1) // vb

    def body(wt_ref, out_ref):
        out_ref[:, 0:hidden] = wt_ref[...].T
        out_ref[:, hidden:128] = jnp.zeros((vb, 128 - hidden), jnp.float32)

    return pl.pallas_call(
        body,
        grid=(grid,),
        in_specs=[pl.BlockSpec((hidden, vb), lambda j: (0, j))],
        out_specs=pl.BlockSpec((vb, 128), lambda j: (j, 0)),
        out_shape=jax.ShapeDtypeStruct((vocab, 128), jnp.float32),
    )(wt)


def _make_gather(batch, seq, hidden):
    total = batch * seq
    n_bt = batch // TILE_B
    n_ht = hidden // 8
    tiles_total = seq * n_bt
    tiles_per_w = tiles_total // NW
    b_per_w = tiles_per_w * TILE_B
    mesh = plsc.VectorSubcoreMesh(core_axis_name="c", subcore_axis_name="s")

    @functools.partial(
        pl.kernel,
        mesh=mesh,
        out_type=jax.ShapeDtypeStruct((seq, n_ht, n_bt, 8, TILE_B), jnp.float32),
        scratch_types=[
            pltpu.VMEM((b_per_w,), jnp.int32),
            pltpu.VMEM((2, TILE_B, hidden), jnp.float32),
            pltpu.VMEM((2, n_ht, 8, TILE_B), jnp.float32),
            pltpu.SemaphoreType.DMA,
            pltpu.SemaphoreType.DMA,
            pltpu.SemaphoreType.DMA,
            pltpu.SemaphoreType.DMA,
        ],
        compiler_params=pltpu.CompilerParams(
            use_tc_tiling_on_sc=False, needs_layout_passes=False),
    )
    def gather_kernel(idx_hbm, table_hbm, out_hbm, idx_v, rows_v, slab_v,
                      sg0, sg1, ss0, ss1):
        wid = lax.axis_index("s") * NC + lax.axis_index("c")
        tile0 = wid * tiles_per_w
        base = wid * b_per_w
        pltpu.sync_copy(idx_hbm.at[pl.ds(base, b_per_w)], idx_v)

        sg = (sg0, sg1)
        ss = (ss0, ss1)
        rows = tuple(rows_v.at[b] for b in range(2))
        slabs = tuple(slab_v.at[b] for b in range(2))
        iota16 = lax.iota(jnp.int32, 16)

        def start_gather(k, b):
            pltpu.async_copy(
                table_hbm.at[idx_v.at[pl.ds(k * TILE_B, TILE_B)]],
                rows[b], sg[b])

        def wait_gather(b):
            pltpu.make_async_copy(
                table_hbm.at[idx_v.at[pl.ds(0, TILE_B)]],
                rows[b], sg[b]).wait()

        def transpose(b):
            def h_body(h, carry):
                ht = h // 8
                hi = h % 8
                col = jnp.full((16,), h, jnp.int32)
                for bq in range(TILE_B // 16):
                    v = plsc.load_gather(rows[b], [bq * 16 + iota16, col])
                    slabs[b][ht, hi, pl.ds(bq * 16, 16)] = v
                return carry
            lax.fori_loop(0, hidden, h_body, 0)

        def start_slab(k, b):
            tid = tile0 + k
            s = tid // n_bt
            bt = tid % n_bt
            pltpu.async_copy(slabs[b], out_hbm.at[s, :, bt], ss[b])

        def wait_slab(b):
            pltpu.make_async_copy(slabs[b], out_hbm.at[0, :, 0], ss[b]).wait()

        start_gather(0, 0)
        start_gather(1, 1)

        def body(j, carry):
            for t in range(2):
                k = 2 * j + t
                b = t
                wait_gather(b)
                transpose(b)

                @pl.when(k + 2 < tiles_per_w)
                def _():
                    start_gather(k + 2, b)

                @pl.when(k >= 2)
                def _():
                    wait_slab(b)

                start_slab(k, b)
            return carry

        lax.fori_loop(0, tiles_per_w // 2, body, 0)
        wait_slab(0)
        wait_slab(1)

    return gather_kernel


def kernel(input_ids, weight):
    batch, seq = input_ids.shape
    vocab, hidden = weight.shape
    total = batch * seq
    w_pad = _transpose_pad(weight).reshape(2 * vocab, hidden)
    flat_ids = (input_ids.T.reshape(total) * 2).astype(jnp.int32)
    out5 = _make_gather(batch, seq, hidden)(flat_ids, w_pad)
    return out5.transpose(2, 4, 0, 1, 3).reshape(batch, seq, hidden)


# speedup vs baseline: 1.9506x; 1.9506x over previous
import functools
import jax
import jax.numpy as jnp
from jax import lax
from jax.experimental import pallas as pl
from jax.experimental.pallas import tpu as pltpu
from jax.experimental.pallas import tpu_sc as plsc

HIDDEN = 64
NC = 2
NS = 16
NW = NC * NS
TILE_B = 128


def _transpose_pad(weight):
    vocab, hidden = weight.shape
    wt = weight.T
    vb = 2048
    grid = (vocab + vb - 1) // vb

    def body(wt_ref, out_ref):
        out_ref[:, 0:hidden] = wt_ref[...].T
        out_ref[:, hidden:128] = jnp.zeros((vb, 128 - hidden), jnp.float32)

    return pl.pallas_call(
        body,
        grid=(grid,),
        in_specs=[pl.BlockSpec((hidden, vb), lambda j: (0, j))],
        out_specs=pl.BlockSpec((vb, 128), lambda j: (j, 0)),
        out_shape=jax.ShapeDtypeStruct((vocab, 128), jnp.float32),
    )(wt)


def _make_gather(batch, seq, hidden):
    total = batch * seq
    n_bt = batch // TILE_B
    n_ht = hidden // 8
    tiles_total = seq * n_bt
    tiles_per_w = tiles_total // NW
    b_per_w = tiles_per_w * TILE_B
    mesh = plsc.VectorSubcoreMesh(core_axis_name="c", subcore_axis_name="s")

    @functools.partial(
        pl.kernel,
        mesh=mesh,
        out_type=jax.ShapeDtypeStruct((seq, n_ht, n_bt, 8, TILE_B), jnp.float32),
        scratch_types=[
            pltpu.VMEM((b_per_w,), jnp.int32),
            pltpu.VMEM((2, TILE_B, hidden), jnp.float32),
            pltpu.VMEM((2, n_ht, 8, TILE_B + 1), jnp.float32),
            pltpu.SemaphoreType.DMA,
            pltpu.SemaphoreType.DMA,
            pltpu.SemaphoreType.DMA,
            pltpu.SemaphoreType.DMA,
        ],
        compiler_params=pltpu.CompilerParams(
            use_tc_tiling_on_sc=False, needs_layout_passes=False),
    )
    def gather_kernel(idx_hbm, table_hbm, out_hbm, idx_v, rows_v, slab_v,
                      sg0, sg1, ss0, ss1):
        wid = lax.axis_index("s") * NC + lax.axis_index("c")
        tile0 = wid * tiles_per_w
        base = wid * b_per_w
        pltpu.sync_copy(idx_hbm.at[pl.ds(base, b_per_w)], idx_v)

        sg = (sg0, sg1)
        ss = (ss0, ss1)
        rows = tuple(rows_v.at[b] for b in range(2))
        slabs = tuple(slab_v.at[b] for b in range(2))
        iota16 = lax.iota(jnp.int32, 16)

        def start_gather(k, b):
            pltpu.async_copy(
                table_hbm.at[idx_v.at[pl.ds(k * TILE_B, TILE_B)]],
                rows[b], sg[b])

        def wait_gather(b):
            pltpu.make_async_copy(
                table_hbm.at[idx_v.at[pl.ds(0, TILE_B)]],
                rows[b], sg[b]).wait()

        ht_base = iota16 // 8
        hi_idx = iota16 % 8

        def transpose(b):
            def b_body(tok, carry):
                for u in range(2):
                    bcol = jnp.full((16,), 2 * tok + u, jnp.int32)
                    for hq in range(hidden // 16):
                        v = rows[b][2 * tok + u, pl.ds(hq * 16, 16)]
                        plsc.store_scatter(
                            slabs[b], [hq * 2 + ht_base, hi_idx, bcol], v)
                return carry
            lax.fori_loop(0, TILE_B // 2, b_body, 0)

        def start_slab(k, b):
            tid = tile0 + k
            s = tid // n_bt
            bt = tid % n_bt
            pltpu.async_copy(
                slabs[b].at[:, :, pl.ds(0, TILE_B)], out_hbm.at[s, :, bt], ss[b])

        def wait_slab(b):
            pltpu.make_async_copy(
                slabs[b].at[:, :, pl.ds(0, TILE_B)], out_hbm.at[0, :, 0],
                ss[b]).wait()

        start_gather(0, 0)
        start_gather(1, 1)

        def body(j, carry):
            for t in range(2):
                k = 2 * j + t
                b = t
                wait_gather(b)
                transpose(b)

                @pl.when(k + 2 < tiles_per_w)
                def _():
                    start_gather(k + 2, b)

                @pl.when(k >= 2)
                def _():
                    wait_slab(b)

                start_slab(k, b)
            return carry

        lax.fori_loop(0, tiles_per_w // 2, body, 0)
        wait_slab(0)
        wait_slab(1)

    return gather_kernel


def kernel(input_ids, weight):
    batch, seq = input_ids.shape
    vocab, hidden = weight.shape
    total = batch * seq
    w_pad = _transpose_pad(weight).reshape(2 * vocab, hidden)
    flat_ids = (input_ids.T.reshape(total) * 2).astype(jnp.int32)
    out5 = _make_gather(batch, seq, hidden)(flat_ids, w_pad)
    return out5.transpose(2, 4, 0, 1, 3).reshape(batch, seq, hidden)


# speedup vs baseline: 2.5238x; 1.2939x over previous
import functools
import jax
import jax.numpy as jnp
from jax import lax
from jax.experimental import pallas as pl
from jax.experimental.pallas import tpu as pltpu
from jax.experimental.pallas import tpu_sc as plsc

HIDDEN = 64
NC = 2
NS = 16
NW = NC * NS
TILE_B = 128


def _transpose_pad(weight):
    vocab, hidden = weight.shape
    wt = weight.T
    vb = 8192
    grid = (vocab + vb - 1) // vb

    def body(wt_ref, out_ref):
        out_ref[:, 0:hidden] = wt_ref[...].T
        out_ref[:, hidden:128] = jnp.zeros((vb, 128 - hidden), jnp.float32)

    return pl.pallas_call(
        body,
        grid=(grid,),
        in_specs=[pl.BlockSpec((hidden, vb), lambda j: (0, j))],
        out_specs=pl.BlockSpec((vb, 128), lambda j: (j, 0)),
        out_shape=jax.ShapeDtypeStruct((vocab, 128), jnp.float32),
    )(wt)


def _make_gather(batch, seq, hidden):
    total = batch * seq
    n_bt = batch // TILE_B
    n_ht = hidden // 8
    tiles_total = seq * n_bt
    tiles_per_w = tiles_total // NW
    b_per_w = tiles_per_w * TILE_B
    mesh = plsc.VectorSubcoreMesh(core_axis_name="c", subcore_axis_name="s")

    @functools.partial(
        pl.kernel,
        mesh=mesh,
        out_type=jax.ShapeDtypeStruct((seq, n_ht, n_bt, 8, TILE_B), jnp.float32),
        scratch_types=[
            pltpu.VMEM((b_per_w,), jnp.int32),
            pltpu.VMEM((2, TILE_B, hidden), jnp.float32),
            pltpu.VMEM((2, n_ht, 8, TILE_B + 1), jnp.float32),
            pltpu.SemaphoreType.DMA,
            pltpu.SemaphoreType.DMA,
            pltpu.SemaphoreType.DMA,
            pltpu.SemaphoreType.DMA,
        ],
        compiler_params=pltpu.CompilerParams(
            use_tc_tiling_on_sc=False, needs_layout_passes=False),
    )
    def gather_kernel(idx_hbm, table_hbm, out_hbm, idx_v, rows_v, slab_v,
                      sg0, sg1, ss0, ss1):
        wid = lax.axis_index("s") * NC + lax.axis_index("c")
        tile0 = wid * tiles_per_w
        base = wid * b_per_w
        pltpu.sync_copy(idx_hbm.at[pl.ds(base, b_per_w)], idx_v)

        sg = (sg0, sg1)
        ss = (ss0, ss1)
        rows = tuple(rows_v.at[b] for b in range(2))
        slabs = tuple(slab_v.at[b] for b in range(2))
        iota16 = lax.iota(jnp.int32, 16)

        def start_gather(k, b):
            pltpu.async_copy(
                table_hbm.at[idx_v.at[pl.ds(k * TILE_B, TILE_B)]],
                rows[b], sg[b])

        def wait_gather(b):
            pltpu.make_async_copy(
                table_hbm.at[idx_v.at[pl.ds(0, TILE_B)]],
                rows[b], sg[b]).wait()

        ht_base = iota16 // 8
        hi_idx = iota16 % 8

        def transpose(b):
            def b_body(tok, carry):
                for u in range(2):
                    bcol = jnp.full((16,), 2 * tok + u, jnp.int32)
                    for hq in range(hidden // 16):
                        v = rows[b][2 * tok + u, pl.ds(hq * 16, 16)]
                        plsc.store_scatter(
                            slabs[b], [hq * 2 + ht_base, hi_idx, bcol], v)
                return carry
            lax.fori_loop(0, TILE_B // 2, b_body, 0)

        def start_slab(k, b):
            tid = tile0 + k
            s = tid // n_bt
            bt = tid % n_bt
            pltpu.async_copy(
                slabs[b].at[:, :, pl.ds(0, TILE_B)], out_hbm.at[s, :, bt], ss[b])

        def wait_slab(b):
            pltpu.make_async_copy(
                slabs[b].at[:, :, pl.ds(0, TILE_B)], out_hbm.at[0, :, 0],
                ss[b]).wait()

        start_gather(0, 0)
        start_gather(1, 1)

        def body(j, carry):
            for t in range(2):
                k = 2 * j + t
                b = t
                wait_gather(b)
                transpose(b)

                @pl.when(k + 2 < tiles_per_w)
                def _():
                    start_gather(k + 2, b)

                @pl.when(k >= 2)
                def _():
                    wait_slab(b)

                start_slab(k, b)
            return carry

        lax.fori_loop(0, tiles_per_w // 2, body, 0)
        wait_slab(0)
        wait_slab(1)

    return gather_kernel


def kernel(input_ids, weight):
    batch, seq = input_ids.shape
    vocab, hidden = weight.shape
    total = batch * seq
    w_pad = _transpose_pad(weight).reshape(2 * vocab, hidden)
    flat_ids = (input_ids.T.reshape(total) * 2).astype(jnp.int32)
    out5 = _make_gather(batch, seq, hidden)(flat_ids, w_pad)
    return out5.transpose(2, 4, 0, 1, 3).reshape(batch, seq, hidden)


# speedup vs baseline: 2.6042x; 1.0318x over previous
import functools
import jax
import jax.numpy as jnp
from jax import lax
from jax.experimental import pallas as pl
from jax.experimental.pallas import tpu as pltpu
from jax.experimental.pallas import tpu_sc as plsc

HIDDEN = 64
NC = 2
NS = 16
NW = NC * NS
TILE_B = 128


def _transpose_pad(weight):
    vocab, hidden = weight.shape
    wt = weight.T
    vb = 16384
    grid = (vocab + vb - 1) // vb

    def body(wt_ref, out_ref):
        out_ref[:, 0:hidden] = wt_ref[...].T
        out_ref[:, hidden:128] = jnp.zeros((vb, 128 - hidden), jnp.float32)

    return pl.pallas_call(
        body,
        grid=(grid,),
        in_specs=[pl.BlockSpec((hidden, vb), lambda j: (0, j))],
        out_specs=pl.BlockSpec((vb, 128), lambda j: (j, 0)),
        out_shape=jax.ShapeDtypeStruct((vocab, 128), jnp.float32),
    )(wt)


def _make_gather(batch, seq, hidden):
    total = batch * seq
    n_bt = batch // TILE_B
    n_ht = hidden // 8
    tiles_total = seq * n_bt
    tiles_per_w = tiles_total // NW
    b_per_w = tiles_per_w * TILE_B
    mesh = plsc.VectorSubcoreMesh(core_axis_name="c", subcore_axis_name="s")

    @functools.partial(
        pl.kernel,
        mesh=mesh,
        out_type=jax.ShapeDtypeStruct((seq, n_ht, n_bt, 8, TILE_B), jnp.float32),
        scratch_types=[
            pltpu.VMEM((b_per_w,), jnp.int32),
            pltpu.VMEM((2, TILE_B, hidden), jnp.float32),
            pltpu.VMEM((2, n_ht, 8, TILE_B + 1), jnp.float32),
            pltpu.SemaphoreType.DMA,
            pltpu.SemaphoreType.DMA,
            pltpu.SemaphoreType.DMA,
            pltpu.SemaphoreType.DMA,
        ],
        compiler_params=pltpu.CompilerParams(
            use_tc_tiling_on_sc=False, needs_layout_passes=False),
    )
    def gather_kernel(idx_hbm, table_hbm, out_hbm, idx_v, rows_v, slab_v,
                      sg0, sg1, ss0, ss1):
        wid = lax.axis_index("s") * NC + lax.axis_index("c")
        tile0 = wid * tiles_per_w
        base = wid * b_per_w
        pltpu.sync_copy(idx_hbm.at[pl.ds(base, b_per_w)], idx_v)

        sg = (sg0, sg1)
        ss = (ss0, ss1)
        rows = tuple(rows_v.at[b] for b in range(2))
        slabs = tuple(slab_v.at[b] for b in range(2))
        iota16 = lax.iota(jnp.int32, 16)

        def start_gather(k, b):
            pltpu.async_copy(
                table_hbm.at[idx_v.at[pl.ds(k * TILE_B, TILE_B)]],
                rows[b], sg[b])

        def wait_gather(b):
            pltpu.make_async_copy(
                table_hbm.at[idx_v.at[pl.ds(0, TILE_B)]],
                rows[b], sg[b]).wait()

        ht_base = iota16 // 8
        hi_idx = iota16 % 8

        def transpose(b):
            def b_body(tok, carry):
                for u in range(2):
                    bcol = jnp.full((16,), 2 * tok + u, jnp.int32)
                    for hq in range(hidden // 16):
                        v = rows[b][2 * tok + u, pl.ds(hq * 16, 16)]
                        plsc.store_scatter(
                            slabs[b], [hq * 2 + ht_base, hi_idx, bcol], v)
                return carry
            lax.fori_loop(0, TILE_B // 2, b_body, 0)

        def start_slab(k, b):
            tid = tile0 + k
            s = tid // n_bt
            bt = tid % n_bt
            pltpu.async_copy(
                slabs[b].at[:, :, pl.ds(0, TILE_B)], out_hbm.at[s, :, bt], ss[b])

        def wait_slab(b):
            pltpu.make_async_copy(
                slabs[b].at[:, :, pl.ds(0, TILE_B)], out_hbm.at[0, :, 0],
                ss[b]).wait()

        start_gather(0, 0)
        start_gather(1, 1)

        def body(j, carry):
            for t in range(2):
                k = 2 * j + t
                b = t
                wait_gather(b)
                transpose(b)

                @pl.when(k + 2 < tiles_per_w)
                def _():
                    start_gather(k + 2, b)

                @pl.when(k >= 2)
                def _():
                    wait_slab(b)

                start_slab(k, b)
            return carry

        lax.fori_loop(0, tiles_per_w // 2, body, 0)
        wait_slab(0)
        wait_slab(1)

    return gather_kernel


def kernel(input_ids, weight):
    batch, seq = input_ids.shape
    vocab, hidden = weight.shape
    total = batch * seq
    w_pad = _transpose_pad(weight).reshape(2 * vocab, hidden)
    flat_ids = (input_ids.T.reshape(total) * 2).astype(jnp.int32)
    out5 = _make_gather(batch, seq, hidden)(flat_ids, w_pad)
    return out5.transpose(2, 4, 0, 1, 3).reshape(batch, seq, hidden)


# speedup vs baseline: 2.6317x; 1.0106x over previous
import functools
import jax
import jax.numpy as jnp
from jax import lax
from jax.experimental import pallas as pl
from jax.experimental.pallas import tpu as pltpu
from jax.experimental.pallas import tpu_sc as plsc

HIDDEN = 64
NC = 2
NS = 16
NW = NC * NS
TILE_B = 128


def _transpose_pad(weight):
    vocab, hidden = weight.shape
    wt = weight.T
    vb = 32768
    grid = (vocab + vb - 1) // vb

    def body(wt_ref, out_ref):
        out_ref[:, 0:hidden] = wt_ref[...].T
        out_ref[:, hidden:128] = jnp.zeros((vb, 128 - hidden), jnp.float32)

    return pl.pallas_call(
        body,
        grid=(grid,),
        in_specs=[pl.BlockSpec((hidden, vb), lambda j: (0, j))],
        out_specs=pl.BlockSpec((vb, 128), lambda j: (j, 0)),
        out_shape=jax.ShapeDtypeStruct((vocab, 128), jnp.float32),
    )(wt)


def _make_gather(batch, seq, hidden):
    total = batch * seq
    n_bt = batch // TILE_B
    n_ht = hidden // 8
    tiles_total = seq * n_bt
    tiles_per_w = tiles_total // NW
    b_per_w = tiles_per_w * TILE_B
    mesh = plsc.VectorSubcoreMesh(core_axis_name="c", subcore_axis_name="s")

    @functools.partial(
        pl.kernel,
        mesh=mesh,
        out_type=jax.ShapeDtypeStruct((seq, n_ht, n_bt, 8, TILE_B), jnp.float32),
        scratch_types=[
            pltpu.VMEM((b_per_w,), jnp.int32),
            pltpu.VMEM((2, TILE_B, hidden), jnp.float32),
            pltpu.VMEM((2, n_ht, 8, TILE_B + 1), jnp.float32),
            pltpu.SemaphoreType.DMA,
            pltpu.SemaphoreType.DMA,
            pltpu.SemaphoreType.DMA,
            pltpu.SemaphoreType.DMA,
        ],
        compiler_params=pltpu.CompilerParams(
            use_tc_tiling_on_sc=False, needs_layout_passes=False),
    )
    def gather_kernel(idx_hbm, table_hbm, out_hbm, idx_v, rows_v, slab_v,
                      sg0, sg1, ss0, ss1):
        wid = lax.axis_index("s") * NC + lax.axis_index("c")
        tile0 = wid * tiles_per_w
        base = wid * b_per_w
        pltpu.sync_copy(idx_hbm.at[pl.ds(base, b_per_w)], idx_v)

        sg = (sg0, sg1)
        ss = (ss0, ss1)
        rows = tuple(rows_v.at[b] for b in range(2))
        slabs = tuple(slab_v.at[b] for b in range(2))
        iota16 = lax.iota(jnp.int32, 16)

        def start_gather(k, b):
            pltpu.async_copy(
                table_hbm.at[idx_v.at[pl.ds(k * TILE_B, TILE_B)]],
                rows[b], sg[b])

        def wait_gather(b):
            pltpu.make_async_copy(
                table_hbm.at[idx_v.at[pl.ds(0, TILE_B)]],
                rows[b], sg[b]).wait()

        ht_base = iota16 // 8
        hi_idx = iota16 % 8

        def transpose(b):
            def b_body(tok, carry):
                for u in range(2):
                    bcol = jnp.full((16,), 2 * tok + u, jnp.int32)
                    for hq in range(hidden // 16):
                        v = rows[b][2 * tok + u, pl.ds(hq * 16, 16)]
                        plsc.store_scatter(
                            slabs[b], [hq * 2 + ht_base, hi_idx, bcol], v)
                return carry
            lax.fori_loop(0, TILE_B // 2, b_body, 0)

        def start_slab(k, b):
            tid = tile0 + k
            s = tid // n_bt
            bt = tid % n_bt
            pltpu.async_copy(
                slabs[b].at[:, :, pl.ds(0, TILE_B)], out_hbm.at[s, :, bt], ss[b])

        def wait_slab(b):
            pltpu.make_async_copy(
                slabs[b].at[:, :, pl.ds(0, TILE_B)], out_hbm.at[0, :, 0],
                ss[b]).wait()

        start_gather(0, 0)
        start_gather(1, 1)

        def body(j, carry):
            for t in range(2):
                k = 2 * j + t
                b = t
                wait_gather(b)
                transpose(b)

                @pl.when(k + 2 < tiles_per_w)
                def _():
                    start_gather(k + 2, b)

                @pl.when(k >= 2)
                def _():
                    wait_slab(b)

                start_slab(k, b)
            return carry

        lax.fori_loop(0, tiles_per_w // 2, body, 0)
        wait_slab(0)
        wait_slab(1)

    return gather_kernel


def kernel(input_ids, weight):
    batch, seq = input_ids.shape
    vocab, hidden = weight.shape
    total = batch * seq
    w_pad = _transpose_pad(weight).reshape(2 * vocab, hidden)
    flat_ids = (input_ids.T.reshape(total) * 2).astype(jnp.int32)
    out5 = _make_gather(batch, seq, hidden)(flat_ids, w_pad)
    return out5.transpose(2, 4, 0, 1, 3).reshape(batch, seq, hidden)


# speedup vs baseline: 2.6808x; 1.0187x over previous
import functools
import jax
import jax.numpy as jnp
from jax import lax
from jax.experimental import pallas as pl
from jax.experimental.pallas import tpu as pltpu
from jax.experimental.pallas import tpu_sc as plsc

HIDDEN = 64
NC = 2
NS = 16
NW = NC * NS
TILE_B = 128


def _transpose_pad(weight):
    vocab, hidden = weight.shape
    wt = weight.T
    vb = 32768
    grid = (vocab + vb - 1) // vb

    def body(wt_ref, out_ref):
        out_ref[:, 0:hidden] = wt_ref[...].T
        out_ref[:, hidden:128] = jnp.zeros((vb, 128 - hidden), jnp.float32)

    return pl.pallas_call(
        body,
        grid=(grid,),
        in_specs=[pl.BlockSpec((hidden, vb), lambda j: (0, j))],
        out_specs=pl.BlockSpec((vb, 128), lambda j: (j, 0)),
        out_shape=jax.ShapeDtypeStruct((vocab, 128), jnp.float32),
    )(wt)


def _make_gather(batch, seq, hidden):
    total = batch * seq
    n_bt = batch // TILE_B
    n_ht = hidden // 8
    tiles_total = seq * n_bt
    tiles_per_w = tiles_total // NW
    b_per_w = tiles_per_w * TILE_B
    mesh = plsc.VectorSubcoreMesh(core_axis_name="c", subcore_axis_name="s")

    @functools.partial(
        pl.kernel,
        mesh=mesh,
        out_type=jax.ShapeDtypeStruct((seq, n_ht, n_bt, 8, TILE_B), jnp.float32),
        scratch_types=[
            pltpu.VMEM((b_per_w,), jnp.int32),
            pltpu.VMEM((2, TILE_B, hidden), jnp.float32),
            pltpu.VMEM((2, n_ht, 8, TILE_B + 1), jnp.float32),
            pltpu.SemaphoreType.DMA,
            pltpu.SemaphoreType.DMA,
            pltpu.SemaphoreType.DMA,
            pltpu.SemaphoreType.DMA,
        ],
        compiler_params=pltpu.CompilerParams(
            use_tc_tiling_on_sc=False, needs_layout_passes=False),
    )
    def gather_kernel(idx_hbm, table_hbm, out_hbm, idx_v, rows_v, slab_v,
                      sg0, sg1, ss0, ss1):
        wid = lax.axis_index("s") * NC + lax.axis_index("c")
        tile0 = wid * tiles_per_w
        base = wid * b_per_w
        pltpu.sync_copy(idx_hbm.at[pl.ds(base, b_per_w)], idx_v)

        sg = (sg0, sg1)
        ss = (ss0, ss1)
        rows = tuple(rows_v.at[b] for b in range(2))
        slabs = tuple(slab_v.at[b] for b in range(2))
        iota16 = lax.iota(jnp.int32, 16)

        def start_gather(k, b):
            pltpu.async_copy(
                table_hbm.at[idx_v.at[pl.ds(k * TILE_B, TILE_B)]],
                rows[b], sg[b])

        def wait_gather(b):
            pltpu.make_async_copy(
                table_hbm.at[idx_v.at[pl.ds(0, TILE_B)]],
                rows[b], sg[b]).wait()

        ht_base = iota16 // 8
        hi_idx = iota16 % 8

        def transpose(b):
            def b_body(tok, carry):
                for u in range(8):
                    bcol = jnp.full((16,), 8 * tok + u, jnp.int32)
                    for hq in range(hidden // 16):
                        v = rows[b][8 * tok + u, pl.ds(hq * 16, 16)]
                        plsc.store_scatter(
                            slabs[b], [hq * 2 + ht_base, hi_idx, bcol], v)
                return carry
            lax.fori_loop(0, TILE_B // 8, b_body, 0)

        def start_slab(k, b):
            tid = tile0 + k
            s = tid // n_bt
            bt = tid % n_bt
            pltpu.async_copy(
                slabs[b].at[:, :, pl.ds(0, TILE_B)], out_hbm.at[s, :, bt], ss[b])

        def wait_slab(b):
            pltpu.make_async_copy(
                slabs[b].at[:, :, pl.ds(0, TILE_B)], out_hbm.at[0, :, 0],
                ss[b]).wait()

        start_gather(0, 0)
        start_gather(1, 1)

        def body(j, carry):
            for t in range(2):
                k = 2 * j + t
                b = t
                wait_gather(b)
                transpose(b)

                @pl.when(k + 2 < tiles_per_w)
                def _():
                    start_gather(k + 2, b)

                @pl.when(k >= 2)
                def _():
                    wait_slab(b)

                start_slab(k, b)
            return carry

        lax.fori_loop(0, tiles_per_w // 2, body, 0)
        wait_slab(0)
        wait_slab(1)

    return gather_kernel


def kernel(input_ids, weight):
    batch, seq = input_ids.shape
    vocab, hidden = weight.shape
    total = batch * seq
    w_pad = _transpose_pad(weight).reshape(2 * vocab, hidden)
    flat_ids = (input_ids.T.reshape(total) * 2).astype(jnp.int32)
    out5 = _make_gather(batch, seq, hidden)(flat_ids, w_pad)
    return out5.transpose(2, 4, 0, 1, 3).reshape(batch, seq, hidden)
